# baseline (speedup 1.0000x reference)
import jax
import jax.numpy as jnp
from jax import lax
from jax.experimental import pallas as pl
from jax.experimental.pallas import tpu as pltpu
from jax.experimental.pallas import tpu_sc as plsc

D_MODEL = 1024
D_FF = 2048
N_EXPERTS = 8
N_TOKENS = 2048
EPS = 1e-8
CLAMP = 10000.0

BT = 128
S_MAX = 2 * N_TOKENS + N_EXPERTS * BT
NT = S_MAX // BT
NTE = NT + 8
FB = 1024
NF = D_FF // FB

GW = 128
NW = 32
TPW = N_TOKENS // NW


def _router_body(x_ref, rw_ref, slot0_ref, slot1_ref, g0_ref, g1_ref, te_ref):
    xv = x_ref[...]
    logits = lax.dot_general(xv, rw_ref[...], (((1,), (1,)), ((), ())))
    logits = jnp.clip(logits, -CLAMP, CLAMP)
    m = jnp.max(logits, axis=-1, keepdims=True)
    e = jnp.exp(logits - m)
    p = e / (jnp.sum(e, axis=-1, keepdims=True) + EPS)
    p = jnp.clip(p, EPS, 1.0)
    iota = lax.broadcasted_iota(jnp.int32, p.shape, 1)
    m1 = jnp.max(p, axis=-1, keepdims=True)
    i1 = jnp.min(jnp.where(p == m1, iota, N_EXPERTS), axis=-1, keepdims=True)
    p2 = jnp.where(iota == i1, -1.0, p)
    m2 = jnp.max(p2, axis=-1, keepdims=True)
    i2 = jnp.min(jnp.where(p2 == m2, iota, N_EXPERTS), axis=-1, keepdims=True)
    denom = m1 + m2
    g0_ref[...] = jnp.broadcast_to(m1 / denom, (N_TOKENS, GW))
    g1_ref[...] = jnp.broadcast_to(m2 / denom, (N_TOKENS, GW))

    a = (jnp.where(iota == i1, 1.0, 0.0) + jnp.where(iota == i2, 1.0, 0.0))
    r_iota = lax.broadcasted_iota(jnp.int32, (N_TOKENS, N_TOKENS), 0)
    c_iota = lax.broadcasted_iota(jnp.int32, (N_TOKENS, N_TOKENS), 1)
    lstrict = jnp.where(c_iota < r_iota, 1.0, 0.0)
    cum = lax.dot_general(lstrict, a, (((1,), (0,)), ((), ())))

    counts = jnp.sum(a, axis=0, keepdims=True)
    ac = jnp.ceil(counts / BT) * BT
    ac_bc = jnp.broadcast_to(ac, (N_TOKENS, N_EXPERTS))

    pos0 = jnp.sum(jnp.where(iota == i1, cum, 0.0), axis=-1, keepdims=True)
    pos1 = jnp.sum(jnp.where(iota == i2, cum, 0.0), axis=-1, keepdims=True)
    base0 = jnp.sum(jnp.where(iota < i1, ac_bc, 0.0), axis=-1, keepdims=True)
    base1 = jnp.sum(jnp.where(iota < i2, ac_bc, 0.0), axis=-1, keepdims=True)
    slot0_ref[...] = (base0 + pos0).astype(jnp.int32)
    slot1_ref[...] = (base1 + pos1).astype(jnp.int32)

    u_r = lax.broadcasted_iota(jnp.int32, (N_EXPERTS, N_EXPERTS), 0)
    u_c = lax.broadcasted_iota(jnp.int32, (N_EXPERTS, N_EXPERTS), 1)
    u = jnp.where(u_r <= u_c, 1.0, 0.0)
    off_end = lax.dot_general(ac, u, (((1,), (0,)), ((), ())))
    off_end_i = off_end.astype(jnp.int32)
    off_end_bc = jnp.broadcast_to(off_end_i, (NTE, N_EXPERTS))
    tile_base = lax.broadcasted_iota(jnp.int32, (NTE, N_EXPERTS), 0) * BT
    te = jnp.sum(jnp.where(off_end_bc <= tile_base, 1, 0), axis=-1, keepdims=True)
    te = jnp.minimum(te, N_EXPERTS - 1)
    n_act = jnp.max(off_end_i, axis=-1, keepdims=True) // BT
    row_i = lax.broadcasted_iota(jnp.int32, (NTE, 1), 0)
    te_ref[...] = jnp.where(row_i < NT, te, jnp.broadcast_to(n_act, (NTE, 1)))


def _router_call(x, router_w):
    return pl.pallas_call(
        _router_body,
        grid=(1,),
        in_specs=[
            pl.BlockSpec((N_TOKENS, D_MODEL), lambda i: (0, 0)),
            pl.BlockSpec((N_EXPERTS, D_MODEL), lambda i: (0, 0)),
        ],
        out_specs=[
            pl.BlockSpec((N_TOKENS, 1), lambda i: (0, 0)),
            pl.BlockSpec((N_TOKENS, 1), lambda i: (0, 0)),
            pl.BlockSpec((N_TOKENS, GW), lambda i: (0, 0)),
            pl.BlockSpec((N_TOKENS, GW), lambda i: (0, 0)),
            pl.BlockSpec((NTE, 1), lambda i: (0, 0)),
        ],
        out_shape=[
            jax.ShapeDtypeStruct((N_TOKENS, 1), jnp.int32),
            jax.ShapeDtypeStruct((N_TOKENS, 1), jnp.int32),
            jax.ShapeDtypeStruct((N_TOKENS, GW), jnp.float32),
            jax.ShapeDtypeStruct((N_TOKENS, GW), jnp.float32),
            jax.ShapeDtypeStruct((NTE, 1), jnp.int32),
        ],
    )(x, router_w)


def _dispatch_body(x_hbm, slot0_hbm, slot1_hbm, g0_hbm, g1_hbm,
                   xd_hbm, gx_hbm,
                   idx0_v, idx1_v, rows_v, ga_v, gb_v, sem, semg):
    wid = lax.axis_index("s") * 2 + lax.axis_index("c")
    base = wid * TPW
    pltpu.sync_copy(slot0_hbm.at[pl.ds(base, TPW)], idx0_v)
    pltpu.sync_copy(slot1_hbm.at[pl.ds(base, TPW)], idx1_v)
    pltpu.sync_copy(x_hbm.at[pl.ds(base, TPW)], rows_v)
    pltpu.sync_copy(g0_hbm.at[pl.ds(base, TPW)], ga_v)
    pltpu.sync_copy(g1_hbm.at[pl.ds(base, TPW)], gb_v)
    cp0 = pltpu.async_copy(rows_v, xd_hbm.at[idx0_v], sem)
    cpg0 = pltpu.async_copy(ga_v, gx_hbm.at[idx0_v], semg)
    cp0.wait()
    cpg0.wait()
    cp1 = pltpu.async_copy(rows_v, xd_hbm.at[idx1_v], sem)
    cpg1 = pltpu.async_copy(gb_v, gx_hbm.at[idx1_v], semg)
    cp1.wait()
    cpg1.wait()


def _dispatch_call(x, slot0, slot1, g0, g1):
    fn = pl.kernel(
        _dispatch_body,
        out_type=[
            jax.ShapeDtypeStruct((S_MAX, D_MODEL), jnp.float32),
            jax.ShapeDtypeStruct((S_MAX, GW), jnp.float32),
        ],
        mesh=plsc.VectorSubcoreMesh(core_axis_name="c", subcore_axis_name="s"),
        scratch_types=[
            pltpu.VMEM((TPW,), jnp.int32),
            pltpu.VMEM((TPW,), jnp.int32),
            pltpu.VMEM((TPW, D_MODEL), jnp.float32),
            pltpu.VMEM((TPW, GW), jnp.float32),
            pltpu.VMEM((TPW, GW), jnp.float32),
            pltpu.SemaphoreType.DMA,
            pltpu.SemaphoreType.DMA,
        ],
    )
    return fn(x, slot0, slot1, g0, g1)


def _ffn_body(te_ref, xd_ref, gx_ref, wg_ref, wu_ref, wd_ref, out_ref):
    i = pl.program_id(0)
    f = pl.program_id(1)
    n_act = te_ref[NT]

    @pl.when(i < n_act)
    def _():
        xv = xd_ref[...]
        gate = lax.dot_general(xv, wg_ref[0, 0], (((1,), (1,)), ((), ())))
        up = lax.dot_general(xv, wu_ref[0, 0], (((1,), (1,)), ((), ())))
        h = (up * jax.nn.sigmoid(up)) * gate
        yp = lax.dot_general(h, wd_ref[0], (((1,), (1,)), ((), ())))

        @pl.when(f == 0)
        def _():
            out_ref[...] = yp

        @pl.when(f != 0)
        def _():
            out_ref[...] += yp

        @pl.when(f == NF - 1)
        def _():
            out_ref[...] *= gx_ref[:, 0:1]


def _ffn_call(te, xd, gx, wg2, w_down):
    def _ieff(i, te_ref):
        return jnp.minimum(i, jnp.maximum(te_ref[NT] - 1, 0))

    return pl.pallas_call(
        _ffn_body,
        grid_spec=pltpu.PrefetchScalarGridSpec(
            num_scalar_prefetch=1,
            grid=(NT, NF),
            in_specs=[
                pl.BlockSpec((BT, D_MODEL),
                             lambda i, f, te_ref: (_ieff(i, te_ref), 0)),
                pl.BlockSpec((BT, GW),
                             lambda i, f, te_ref: (_ieff(i, te_ref), 0)),
                pl.BlockSpec((1, 1, FB, D_MODEL),
                             lambda i, f, te_ref: (te_ref[_ieff(i, te_ref)], 0, f, 0)),
                pl.BlockSpec((1, 1, FB, D_MODEL),
                             lambda i, f, te_ref: (te_ref[_ieff(i, te_ref)], 1, f, 0)),
                pl.BlockSpec((1, D_MODEL, FB),
                             lambda i, f, te_ref: (te_ref[_ieff(i, te_ref)], 0, f)),
            ],
            out_specs=pl.BlockSpec((BT, D_MODEL), lambda i, f, te_ref: (i, 0)),
        ),
        out_shape=jax.ShapeDtypeStruct((S_MAX, D_MODEL), jnp.float32),
        compiler_params=pltpu.CompilerParams(
            dimension_semantics=("arbitrary", "arbitrary"),
        ),
    )(te, xd, gx, wg2, wg2, w_down)


CH = 32


def _combine_body(ys_hbm, slot0_hbm, slot1_hbm, y_hbm,
                  idx0_v, idx1_v, rows_a, rows_b, sem_a, sem_b):
    wid = lax.axis_index("s") * 2 + lax.axis_index("c")

    def chunk(c, _):
        base = wid * TPW + c * CH
        pltpu.sync_copy(slot0_hbm.at[pl.ds(base, CH)], idx0_v)
        pltpu.sync_copy(slot1_hbm.at[pl.ds(base, CH)], idx1_v)
        cpa = pltpu.async_copy(ys_hbm.at[idx0_v], rows_a, sem_a)
        cpb = pltpu.async_copy(ys_hbm.at[idx1_v], rows_b, sem_b)
        cpa.wait()
        cpb.wait()

        def row(r, _):
            def col(j, _):
                rows_a[r, pl.ds(j * 16, 16)] = (
                    rows_a[r, pl.ds(j * 16, 16)] + rows_b[r, pl.ds(j * 16, 16)])
                return 0

            lax.fori_loop(0, D_MODEL // 16, col, 0, unroll=8)
            return 0

        lax.fori_loop(0, CH, row, 0)
        pltpu.sync_copy(rows_a, y_hbm.at[pl.ds(base, CH)])
        return 0

    lax.fori_loop(0, TPW // CH, chunk, 0)


def _combine_call(ys, slot0, slot1):
    fn = pl.kernel(
        _combine_body,
        out_type=jax.ShapeDtypeStruct((N_TOKENS, D_MODEL), jnp.float32),
        mesh=plsc.VectorSubcoreMesh(core_axis_name="c", subcore_axis_name="s"),
        scratch_types=[
            pltpu.VMEM((CH,), jnp.int32),
            pltpu.VMEM((CH,), jnp.int32),
            pltpu.VMEM((CH, D_MODEL), jnp.float32),
            pltpu.VMEM((CH, D_MODEL), jnp.float32),
            pltpu.SemaphoreType.DMA,
            pltpu.SemaphoreType.DMA,
        ],
    )
    return fn(ys, slot0, slot1)


def kernel(x, router_w, w_gate_up, w_down):
    wg2 = w_gate_up.reshape(N_EXPERTS, 2, D_FF, D_MODEL)
    slot0, slot1, g0, g1, te = _router_call(x, router_w)
    slot0 = slot0.reshape(N_TOKENS)
    slot1 = slot1.reshape(N_TOKENS)
    te = te.reshape(NTE)
    xd, gx = _dispatch_call(x, slot0, slot1, g0, g1)
    ys = _ffn_call(te, xd, gx, wg2, w_down)
    return _combine_call(ys, slot0, slot1)


# speedup vs baseline: 1.4749x; 1.4749x over previous
import jax
import jax.numpy as jnp
from jax import lax
from jax.experimental import pallas as pl
from jax.experimental.pallas import tpu as pltpu
from jax.experimental.pallas import tpu_sc as plsc

D_MODEL = 1024
D_FF = 2048
N_EXPERTS = 8
N_TOKENS = 2048
EPS = 1e-8
CLAMP = 10000.0

BT = 256
S_MAX = 2 * N_TOKENS + N_EXPERTS * BT
NT = S_MAX // BT
NTE = NT + 8
FB = 1024
NF = D_FF // FB

GW = 128
NW = 32
TPW = N_TOKENS // NW


def _router_body(x_ref, rw_ref, slot0_ref, slot1_ref, g0_ref, g1_ref, te_ref):
    xv = x_ref[...]
    logits = lax.dot_general(xv, rw_ref[...], (((1,), (1,)), ((), ())))
    logits = jnp.clip(logits, -CLAMP, CLAMP)
    m = jnp.max(logits, axis=-1, keepdims=True)
    e = jnp.exp(logits - m)
    p = e / (jnp.sum(e, axis=-1, keepdims=True) + EPS)
    p = jnp.clip(p, EPS, 1.0)
    iota = lax.broadcasted_iota(jnp.int32, p.shape, 1)
    m1 = jnp.max(p, axis=-1, keepdims=True)
    i1 = jnp.min(jnp.where(p == m1, iota, N_EXPERTS), axis=-1, keepdims=True)
    p2 = jnp.where(iota == i1, -1.0, p)
    m2 = jnp.max(p2, axis=-1, keepdims=True)
    i2 = jnp.min(jnp.where(p2 == m2, iota, N_EXPERTS), axis=-1, keepdims=True)
    denom = m1 + m2
    g0_ref[...] = jnp.broadcast_to(m1 / denom, (N_TOKENS, GW))
    g1_ref[...] = jnp.broadcast_to(m2 / denom, (N_TOKENS, GW))

    a = (jnp.where(iota == i1, 1.0, 0.0) + jnp.where(iota == i2, 1.0, 0.0))
    r_iota = lax.broadcasted_iota(jnp.int32, (N_TOKENS, N_TOKENS), 0)
    c_iota = lax.broadcasted_iota(jnp.int32, (N_TOKENS, N_TOKENS), 1)
    lstrict = jnp.where(c_iota < r_iota, 1.0, 0.0)
    cum = lax.dot_general(lstrict, a, (((1,), (0,)), ((), ())))

    counts = jnp.sum(a, axis=0, keepdims=True)
    ac = jnp.ceil(counts / BT) * BT
    ac_bc = jnp.broadcast_to(ac, (N_TOKENS, N_EXPERTS))

    pos0 = jnp.sum(jnp.where(iota == i1, cum, 0.0), axis=-1, keepdims=True)
    pos1 = jnp.sum(jnp.where(iota == i2, cum, 0.0), axis=-1, keepdims=True)
    base0 = jnp.sum(jnp.where(iota < i1, ac_bc, 0.0), axis=-1, keepdims=True)
    base1 = jnp.sum(jnp.where(iota < i2, ac_bc, 0.0), axis=-1, keepdims=True)
    slot0_ref[...] = (base0 + pos0).astype(jnp.int32)
    slot1_ref[...] = (base1 + pos1).astype(jnp.int32)

    u_r = lax.broadcasted_iota(jnp.int32, (N_EXPERTS, N_EXPERTS), 0)
    u_c = lax.broadcasted_iota(jnp.int32, (N_EXPERTS, N_EXPERTS), 1)
    u = jnp.where(u_r <= u_c, 1.0, 0.0)
    off_end = lax.dot_general(ac, u, (((1,), (0,)), ((), ())))
    off_end_i = off_end.astype(jnp.int32)
    off_end_bc = jnp.broadcast_to(off_end_i, (NTE, N_EXPERTS))
    tile_base = lax.broadcasted_iota(jnp.int32, (NTE, N_EXPERTS), 0) * BT
    te = jnp.sum(jnp.where(off_end_bc <= tile_base, 1, 0), axis=-1, keepdims=True)
    te = jnp.minimum(te, N_EXPERTS - 1)
    n_act = jnp.max(off_end_i, axis=-1, keepdims=True) // BT
    row_i = lax.broadcasted_iota(jnp.int32, (NTE, 1), 0)
    te_ref[...] = jnp.where(row_i < NT, te, jnp.broadcast_to(n_act, (NTE, 1)))


def _router_call(x, router_w):
    return pl.pallas_call(
        _router_body,
        grid=(1,),
        in_specs=[
            pl.BlockSpec((N_TOKENS, D_MODEL), lambda i: (0, 0)),
            pl.BlockSpec((N_EXPERTS, D_MODEL), lambda i: (0, 0)),
        ],
        out_specs=[
            pl.BlockSpec((N_TOKENS, 1), lambda i: (0, 0)),
            pl.BlockSpec((N_TOKENS, 1), lambda i: (0, 0)),
            pl.BlockSpec((N_TOKENS, GW), lambda i: (0, 0)),
            pl.BlockSpec((N_TOKENS, GW), lambda i: (0, 0)),
            pl.BlockSpec((NTE, 1), lambda i: (0, 0)),
        ],
        out_shape=[
            jax.ShapeDtypeStruct((N_TOKENS, 1), jnp.int32),
            jax.ShapeDtypeStruct((N_TOKENS, 1), jnp.int32),
            jax.ShapeDtypeStruct((N_TOKENS, GW), jnp.float32),
            jax.ShapeDtypeStruct((N_TOKENS, GW), jnp.float32),
            jax.ShapeDtypeStruct((NTE, 1), jnp.int32),
        ],
    )(x, router_w)


def _dispatch_body(x_hbm, slot0_hbm, slot1_hbm, g0_hbm, g1_hbm,
                   xd_hbm, gx_hbm,
                   idx0_v, idx1_v, rows_v, ga_v, gb_v, sem, semg):
    wid = lax.axis_index("s") * 2 + lax.axis_index("c")
    base = wid * TPW
    l0 = pltpu.async_copy(slot0_hbm.at[pl.ds(base, TPW)], idx0_v, sem)
    l1 = pltpu.async_copy(slot1_hbm.at[pl.ds(base, TPW)], idx1_v, sem)
    l2 = pltpu.async_copy(x_hbm.at[pl.ds(base, TPW)], rows_v, sem)
    l3 = pltpu.async_copy(g0_hbm.at[pl.ds(base, TPW)], ga_v, sem)
    l4 = pltpu.async_copy(g1_hbm.at[pl.ds(base, TPW)], gb_v, sem)
    l0.wait(); l1.wait(); l2.wait(); l3.wait(); l4.wait()
    cp0 = pltpu.async_copy(rows_v, xd_hbm.at[idx0_v], sem)
    cpg0 = pltpu.async_copy(ga_v, gx_hbm.at[idx0_v], semg)
    cp1 = pltpu.async_copy(rows_v, xd_hbm.at[idx1_v], sem)
    cpg1 = pltpu.async_copy(gb_v, gx_hbm.at[idx1_v], semg)
    cp0.wait(); cpg0.wait(); cp1.wait(); cpg1.wait()


def _dispatch_call(x, slot0, slot1, g0, g1):
    fn = pl.kernel(
        _dispatch_body,
        out_type=[
            jax.ShapeDtypeStruct((S_MAX, D_MODEL), jnp.float32),
            jax.ShapeDtypeStruct((S_MAX, GW), jnp.float32),
        ],
        mesh=plsc.VectorSubcoreMesh(core_axis_name="c", subcore_axis_name="s"),
        scratch_types=[
            pltpu.VMEM((TPW,), jnp.int32),
            pltpu.VMEM((TPW,), jnp.int32),
            pltpu.VMEM((TPW, D_MODEL), jnp.float32),
            pltpu.VMEM((TPW, GW), jnp.float32),
            pltpu.VMEM((TPW, GW), jnp.float32),
            pltpu.SemaphoreType.DMA,
            pltpu.SemaphoreType.DMA,
        ],
    )
    return fn(x, slot0, slot1, g0, g1)


def _ffn_body(te_ref, xd_ref, gx_ref, wg_ref, wu_ref, wd_ref, out_ref):
    i = pl.program_id(0)
    f = pl.program_id(1)
    n_act = te_ref[NT]

    @pl.when(i < n_act)
    def _():
        xv = xd_ref[...]
        gate = lax.dot_general(xv, wg_ref[0, 0], (((1,), (1,)), ((), ())))
        up = lax.dot_general(xv, wu_ref[0, 0], (((1,), (1,)), ((), ())))
        h = (up * jax.nn.sigmoid(up)) * gate
        yp = lax.dot_general(h, wd_ref[0], (((1,), (1,)), ((), ())))

        @pl.when(f == 0)
        def _():
            out_ref[...] = yp

        @pl.when(f != 0)
        def _():
            out_ref[...] += yp

        @pl.when(f == NF - 1)
        def _():
            out_ref[...] *= gx_ref[:, 0:1]


def _ffn_call(te, xd, gx, wg2, w_down):
    def _ieff(i, te_ref):
        return jnp.minimum(i, jnp.maximum(te_ref[NT] - 1, 0))

    return pl.pallas_call(
        _ffn_body,
        grid_spec=pltpu.PrefetchScalarGridSpec(
            num_scalar_prefetch=1,
            grid=(NT, NF),
            in_specs=[
                pl.BlockSpec((BT, D_MODEL),
                             lambda i, f, te_ref: (_ieff(i, te_ref), 0)),
                pl.BlockSpec((BT, GW),
                             lambda i, f, te_ref: (_ieff(i, te_ref), 0)),
                pl.BlockSpec((1, 1, FB, D_MODEL),
                             lambda i, f, te_ref: (te_ref[_ieff(i, te_ref)], 0, f, 0)),
                pl.BlockSpec((1, 1, FB, D_MODEL),
                             lambda i, f, te_ref: (te_ref[_ieff(i, te_ref)], 1, f, 0)),
                pl.BlockSpec((1, D_MODEL, FB),
                             lambda i, f, te_ref: (te_ref[_ieff(i, te_ref)], 0, f)),
            ],
            out_specs=pl.BlockSpec((BT, D_MODEL), lambda i, f, te_ref: (i, 0)),
        ),
        out_shape=jax.ShapeDtypeStruct((S_MAX, D_MODEL), jnp.float32),
        compiler_params=pltpu.CompilerParams(
            dimension_semantics=("arbitrary", "arbitrary"),
        ),
    )(te, xd, gx, wg2, wg2, w_down)


CH = 32


def _combine_body(ys_hbm, slot0_hbm, slot1_hbm, y_hbm,
                  idx0_v, idx1_v, rows_a, rows_b, sem_a, sem_b):
    wid = lax.axis_index("s") * 2 + lax.axis_index("c")

    def chunk(c, _):
        base = wid * TPW + c * CH
        pltpu.sync_copy(slot0_hbm.at[pl.ds(base, CH)], idx0_v)
        pltpu.sync_copy(slot1_hbm.at[pl.ds(base, CH)], idx1_v)
        cpa = pltpu.async_copy(ys_hbm.at[idx0_v], rows_a, sem_a)
        cpb = pltpu.async_copy(ys_hbm.at[idx1_v], rows_b, sem_b)
        cpa.wait()
        cpb.wait()

        @plsc.parallel_loop(0, CH, 1, unroll=2)
        def _(r):
            for j in range(D_MODEL // 16):
                rows_a[r, pl.ds(j * 16, 16)] = (
                    rows_a[r, pl.ds(j * 16, 16)] + rows_b[r, pl.ds(j * 16, 16)])

        pltpu.sync_copy(rows_a, y_hbm.at[pl.ds(base, CH)])
        return 0

    lax.fori_loop(0, TPW // CH, chunk, 0)


def _combine_call(ys, slot0, slot1):
    fn = pl.kernel(
        _combine_body,
        out_type=jax.ShapeDtypeStruct((N_TOKENS, D_MODEL), jnp.float32),
        mesh=plsc.VectorSubcoreMesh(core_axis_name="c", subcore_axis_name="s"),
        scratch_types=[
            pltpu.VMEM((CH,), jnp.int32),
            pltpu.VMEM((CH,), jnp.int32),
            pltpu.VMEM((CH, D_MODEL), jnp.float32),
            pltpu.VMEM((CH, D_MODEL), jnp.float32),
            pltpu.SemaphoreType.DMA,
            pltpu.SemaphoreType.DMA,
        ],
    )
    return fn(ys, slot0, slot1)


def kernel(x, router_w, w_gate_up, w_down):
    wg2 = w_gate_up.reshape(N_EXPERTS, 2, D_FF, D_MODEL)
    slot0, slot1, g0, g1, te = _router_call(x, router_w)
    slot0 = slot0.reshape(N_TOKENS)
    slot1 = slot1.reshape(N_TOKENS)
    te = te.reshape(NTE)
    xd, gx = _dispatch_call(x, slot0, slot1, g0, g1)
    ys = _ffn_call(te, xd, gx, wg2, w_down)
    return _combine_call(ys, slot0, slot1)


# speedup vs baseline: 2.0724x; 1.4051x over previous
import jax
import jax.numpy as jnp
from jax import lax
from jax.experimental import pallas as pl
from jax.experimental.pallas import tpu as pltpu
from jax.experimental.pallas import tpu_sc as plsc

D_MODEL = 1024
D_FF = 2048
N_EXPERTS = 8
N_TOKENS = 2048
EPS = 1e-8
CLAMP = 10000.0

BT = 256
S_MAX = 2 * N_TOKENS + N_EXPERTS * BT
NT = S_MAX // BT
NTE = NT + 8
FB = 1024
NF = D_FF // FB

GW = 128
NW = 32
TPW = N_TOKENS // NW


def _router_body(x_ref, rw_ref, slot0_ref, slot1_ref, g0_ref, g1_ref, te_ref):
    xv = x_ref[...]
    logits = lax.dot_general(xv, rw_ref[...], (((1,), (1,)), ((), ())))
    logits = jnp.clip(logits, -CLAMP, CLAMP)
    m = jnp.max(logits, axis=-1, keepdims=True)
    e = jnp.exp(logits - m)
    p = e / (jnp.sum(e, axis=-1, keepdims=True) + EPS)
    p = jnp.clip(p, EPS, 1.0)
    iota = lax.broadcasted_iota(jnp.int32, p.shape, 1)
    m1 = jnp.max(p, axis=-1, keepdims=True)
    i1 = jnp.min(jnp.where(p == m1, iota, N_EXPERTS), axis=-1, keepdims=True)
    p2 = jnp.where(iota == i1, -1.0, p)
    m2 = jnp.max(p2, axis=-1, keepdims=True)
    i2 = jnp.min(jnp.where(p2 == m2, iota, N_EXPERTS), axis=-1, keepdims=True)
    denom = m1 + m2
    g0_ref[...] = jnp.broadcast_to(m1 / denom, (N_TOKENS, GW))
    g1_ref[...] = jnp.broadcast_to(m2 / denom, (N_TOKENS, GW))

    a = (jnp.where(iota == i1, 1.0, 0.0) + jnp.where(iota == i2, 1.0, 0.0))
    r_iota = lax.broadcasted_iota(jnp.int32, (N_TOKENS, N_TOKENS), 0)
    c_iota = lax.broadcasted_iota(jnp.int32, (N_TOKENS, N_TOKENS), 1)
    lstrict = jnp.where(c_iota < r_iota, 1.0, 0.0)
    cum = lax.dot_general(lstrict, a, (((1,), (0,)), ((), ())))

    counts = jnp.sum(a, axis=0, keepdims=True)
    ac = jnp.ceil(counts / BT) * BT
    ac_bc = jnp.broadcast_to(ac, (N_TOKENS, N_EXPERTS))

    pos0 = jnp.sum(jnp.where(iota == i1, cum, 0.0), axis=-1, keepdims=True)
    pos1 = jnp.sum(jnp.where(iota == i2, cum, 0.0), axis=-1, keepdims=True)
    base0 = jnp.sum(jnp.where(iota < i1, ac_bc, 0.0), axis=-1, keepdims=True)
    base1 = jnp.sum(jnp.where(iota < i2, ac_bc, 0.0), axis=-1, keepdims=True)
    slot0_ref[...] = (base0 + pos0).astype(jnp.int32)
    slot1_ref[...] = (base1 + pos1).astype(jnp.int32)

    u_r = lax.broadcasted_iota(jnp.int32, (N_EXPERTS, N_EXPERTS), 0)
    u_c = lax.broadcasted_iota(jnp.int32, (N_EXPERTS, N_EXPERTS), 1)
    u = jnp.where(u_r <= u_c, 1.0, 0.0)
    off_end = lax.dot_general(ac, u, (((1,), (0,)), ((), ())))
    off_end_i = off_end.astype(jnp.int32)
    off_end_bc = jnp.broadcast_to(off_end_i, (NTE, N_EXPERTS))
    tile_base = lax.broadcasted_iota(jnp.int32, (NTE, N_EXPERTS), 0) * BT
    te = jnp.sum(jnp.where(off_end_bc <= tile_base, 1, 0), axis=-1, keepdims=True)
    te = jnp.minimum(te, N_EXPERTS - 1)
    n_act = jnp.max(off_end_i, axis=-1, keepdims=True) // BT
    row_i = lax.broadcasted_iota(jnp.int32, (NTE, 1), 0)
    te_ref[...] = jnp.where(row_i < NT, te, jnp.broadcast_to(n_act, (NTE, 1)))


def _router_call(x, router_w):
    return pl.pallas_call(
        _router_body,
        grid=(1,),
        in_specs=[
            pl.BlockSpec((N_TOKENS, D_MODEL), lambda i: (0, 0)),
            pl.BlockSpec((N_EXPERTS, D_MODEL), lambda i: (0, 0)),
        ],
        out_specs=[
            pl.BlockSpec((N_TOKENS, 1), lambda i: (0, 0)),
            pl.BlockSpec((N_TOKENS, 1), lambda i: (0, 0)),
            pl.BlockSpec((N_TOKENS, GW), lambda i: (0, 0)),
            pl.BlockSpec((N_TOKENS, GW), lambda i: (0, 0)),
            pl.BlockSpec((NTE, 1), lambda i: (0, 0)),
        ],
        out_shape=[
            jax.ShapeDtypeStruct((N_TOKENS, 1), jnp.int32),
            jax.ShapeDtypeStruct((N_TOKENS, 1), jnp.int32),
            jax.ShapeDtypeStruct((N_TOKENS, GW), jnp.float32),
            jax.ShapeDtypeStruct((N_TOKENS, GW), jnp.float32),
            jax.ShapeDtypeStruct((NTE, 1), jnp.int32),
        ],
    )(x, router_w)


def _dispatch_body(x_hbm, slot0_hbm, slot1_hbm, g0_hbm, g1_hbm,
                   xd_hbm, gx_hbm,
                   idx0_v, idx1_v, rows_v, ga_v, gb_v, sem, semg):
    wid = lax.axis_index("s") * 2 + lax.axis_index("c")
    base = wid * TPW
    l0 = pltpu.async_copy(slot0_hbm.at[pl.ds(base, TPW)], idx0_v, sem)
    l1 = pltpu.async_copy(slot1_hbm.at[pl.ds(base, TPW)], idx1_v, sem)
    l2 = pltpu.async_copy(x_hbm.at[pl.ds(base, TPW)], rows_v, sem)
    l3 = pltpu.async_copy(g0_hbm.at[pl.ds(base, TPW)], ga_v, sem)
    l4 = pltpu.async_copy(g1_hbm.at[pl.ds(base, TPW)], gb_v, sem)
    l0.wait(); l1.wait(); l2.wait(); l3.wait(); l4.wait()
    cp0 = pltpu.async_copy(rows_v, xd_hbm.at[idx0_v], sem)
    cpg0 = pltpu.async_copy(ga_v, gx_hbm.at[idx0_v], semg)
    cp1 = pltpu.async_copy(rows_v, xd_hbm.at[idx1_v], sem)
    cpg1 = pltpu.async_copy(gb_v, gx_hbm.at[idx1_v], semg)
    cp0.wait(); cpg0.wait(); cp1.wait(); cpg1.wait()


def _dispatch_call(x, slot0, slot1, g0, g1):
    fn = pl.kernel(
        _dispatch_body,
        out_type=[
            jax.ShapeDtypeStruct((S_MAX, D_MODEL), jnp.float32),
            jax.ShapeDtypeStruct((S_MAX, GW), jnp.float32),
        ],
        mesh=plsc.VectorSubcoreMesh(core_axis_name="c", subcore_axis_name="s"),
        scratch_types=[
            pltpu.VMEM((TPW,), jnp.int32),
            pltpu.VMEM((TPW,), jnp.int32),
            pltpu.VMEM((TPW, D_MODEL), jnp.float32),
            pltpu.VMEM((TPW, GW), jnp.float32),
            pltpu.VMEM((TPW, GW), jnp.float32),
            pltpu.SemaphoreType.DMA,
            pltpu.SemaphoreType.DMA,
        ],
    )
    return fn(x, slot0, slot1, g0, g1)


def _ffn_body(te_ref, xd_ref, gx_ref, wg_ref, wu_ref, wd_ref, out_ref):
    i = pl.program_id(0)
    n_act = te_ref[NT]

    @pl.when(i < n_act)
    def _():
        xv = xd_ref[...]
        gate = lax.dot_general(xv, wg_ref[0, 0], (((1,), (1,)), ((), ())))
        up = lax.dot_general(xv, wu_ref[0, 0], (((1,), (1,)), ((), ())))
        h = (up * jax.nn.sigmoid(up)) * gate
        yp = lax.dot_general(h, wd_ref[0], (((1,), (1,)), ((), ())))
        out_ref[...] = yp * gx_ref[:, 0:1]


def _ffn_call(te, xd, gx, wg2, w_down):
    def _ieff(i, te_ref):
        return jnp.minimum(i, jnp.maximum(te_ref[NT] - 1, 0))

    return pl.pallas_call(
        _ffn_body,
        grid_spec=pltpu.PrefetchScalarGridSpec(
            num_scalar_prefetch=1,
            grid=(NT,),
            in_specs=[
                pl.BlockSpec((BT, D_MODEL),
                             lambda i, te_ref: (_ieff(i, te_ref), 0)),
                pl.BlockSpec((BT, GW),
                             lambda i, te_ref: (_ieff(i, te_ref), 0)),
                pl.BlockSpec((1, 1, D_FF, D_MODEL),
                             lambda i, te_ref: (te_ref[_ieff(i, te_ref)], 0, 0, 0)),
                pl.BlockSpec((1, 1, D_FF, D_MODEL),
                             lambda i, te_ref: (te_ref[_ieff(i, te_ref)], 1, 0, 0)),
                pl.BlockSpec((1, D_MODEL, D_FF),
                             lambda i, te_ref: (te_ref[_ieff(i, te_ref)], 0, 0)),
            ],
            out_specs=pl.BlockSpec((BT, D_MODEL), lambda i, te_ref: (i, 0)),
        ),
        out_shape=jax.ShapeDtypeStruct((S_MAX, D_MODEL), jnp.float32),
        compiler_params=pltpu.CompilerParams(
            dimension_semantics=("arbitrary",),
            vmem_limit_bytes=104 * 1024 * 1024,
        ),
    )(te, xd, gx, wg2, wg2, w_down)


CH = 32


def _combine_body(ys_hbm, slot0_hbm, slot1_hbm, y_hbm,
                  idx0_v, idx1_v, rows_a, rows_b, sem_a, sem_b):
    wid = lax.axis_index("s") * 2 + lax.axis_index("c")

    def chunk(c, _):
        base = wid * TPW + c * CH
        pltpu.sync_copy(slot0_hbm.at[pl.ds(base, CH)], idx0_v)
        pltpu.sync_copy(slot1_hbm.at[pl.ds(base, CH)], idx1_v)
        cpa = pltpu.async_copy(ys_hbm.at[idx0_v], rows_a, sem_a)
        cpb = pltpu.async_copy(ys_hbm.at[idx1_v], rows_b, sem_b)
        cpa.wait()
        cpb.wait()

        @plsc.parallel_loop(0, CH, 1, unroll=2)
        def _(r):
            for j in range(D_MODEL // 16):
                rows_a[r, pl.ds(j * 16, 16)] = (
                    rows_a[r, pl.ds(j * 16, 16)] + rows_b[r, pl.ds(j * 16, 16)])

        pltpu.sync_copy(rows_a, y_hbm.at[pl.ds(base, CH)])
        return 0

    lax.fori_loop(0, TPW // CH, chunk, 0)


def _combine_call(ys, slot0, slot1):
    fn = pl.kernel(
        _combine_body,
        out_type=jax.ShapeDtypeStruct((N_TOKENS, D_MODEL), jnp.float32),
        mesh=plsc.VectorSubcoreMesh(core_axis_name="c", subcore_axis_name="s"),
        scratch_types=[
            pltpu.VMEM((CH,), jnp.int32),
            pltpu.VMEM((CH,), jnp.int32),
            pltpu.VMEM((CH, D_MODEL), jnp.float32),
            pltpu.VMEM((CH, D_MODEL), jnp.float32),
            pltpu.SemaphoreType.DMA,
            pltpu.SemaphoreType.DMA,
        ],
    )
    return fn(ys, slot0, slot1)


def kernel(x, router_w, w_gate_up, w_down):
    wg2 = w_gate_up.reshape(N_EXPERTS, 2, D_FF, D_MODEL)
    slot0, slot1, g0, g1, te = _router_call(x, router_w)
    slot0 = slot0.reshape(N_TOKENS)
    slot1 = slot1.reshape(N_TOKENS)
    te = te.reshape(NTE)
    xd, gx = _dispatch_call(x, slot0, slot1, g0, g1)
    ys = _ffn_call(te, xd, gx, wg2, w_down)
    return _combine_call(ys, slot0, slot1)


# speedup vs baseline: 2.2730x; 1.0968x over previous
import jax
import jax.numpy as jnp
from jax import lax
from jax.experimental import pallas as pl
from jax.experimental.pallas import tpu as pltpu
from jax.experimental.pallas import tpu_sc as plsc

D_MODEL = 1024
D_FF = 2048
N_EXPERTS = 8
N_TOKENS = 2048
EPS = 1e-8
CLAMP = 10000.0

BT = 512
S_MAX = 2 * N_TOKENS + N_EXPERTS * BT
NT = S_MAX // BT
NTE = NT + 8
FB = 1024
NF = D_FF // FB

GW = 128
NW = 32
TPW = N_TOKENS // NW


def _router_body(x_ref, rw_ref, slot0_ref, slot1_ref, g0_ref, g1_ref, te_ref):
    xv = x_ref[...]
    logits = lax.dot_general(xv, rw_ref[...], (((1,), (1,)), ((), ())))
    logits = jnp.clip(logits, -CLAMP, CLAMP)
    m = jnp.max(logits, axis=-1, keepdims=True)
    e = jnp.exp(logits - m)
    p = e / (jnp.sum(e, axis=-1, keepdims=True) + EPS)
    p = jnp.clip(p, EPS, 1.0)
    iota = lax.broadcasted_iota(jnp.int32, p.shape, 1)
    m1 = jnp.max(p, axis=-1, keepdims=True)
    i1 = jnp.min(jnp.where(p == m1, iota, N_EXPERTS), axis=-1, keepdims=True)
    p2 = jnp.where(iota == i1, -1.0, p)
    m2 = jnp.max(p2, axis=-1, keepdims=True)
    i2 = jnp.min(jnp.where(p2 == m2, iota, N_EXPERTS), axis=-1, keepdims=True)
    denom = m1 + m2
    g0_ref[...] = jnp.broadcast_to(m1 / denom, (N_TOKENS, GW))
    g1_ref[...] = jnp.broadcast_to(m2 / denom, (N_TOKENS, GW))

    a = (jnp.where(iota == i1, 1.0, 0.0) + jnp.where(iota == i2, 1.0, 0.0))
    r_iota = lax.broadcasted_iota(jnp.int32, (N_TOKENS, N_TOKENS), 0)
    c_iota = lax.broadcasted_iota(jnp.int32, (N_TOKENS, N_TOKENS), 1)
    lstrict = jnp.where(c_iota < r_iota, 1.0, 0.0)
    cum = lax.dot_general(lstrict, a, (((1,), (0,)), ((), ())))

    counts = jnp.sum(a, axis=0, keepdims=True)
    ac = jnp.ceil(counts / BT) * BT
    ac_bc = jnp.broadcast_to(ac, (N_TOKENS, N_EXPERTS))

    pos0 = jnp.sum(jnp.where(iota == i1, cum, 0.0), axis=-1, keepdims=True)
    pos1 = jnp.sum(jnp.where(iota == i2, cum, 0.0), axis=-1, keepdims=True)
    base0 = jnp.sum(jnp.where(iota < i1, ac_bc, 0.0), axis=-1, keepdims=True)
    base1 = jnp.sum(jnp.where(iota < i2, ac_bc, 0.0), axis=-1, keepdims=True)
    slot0_ref[...] = (base0 + pos0).astype(jnp.int32)
    slot1_ref[...] = (base1 + pos1).astype(jnp.int32)

    u_r = lax.broadcasted_iota(jnp.int32, (N_EXPERTS, N_EXPERTS), 0)
    u_c = lax.broadcasted_iota(jnp.int32, (N_EXPERTS, N_EXPERTS), 1)
    u = jnp.where(u_r <= u_c, 1.0, 0.0)
    off_end = lax.dot_general(ac, u, (((1,), (0,)), ((), ())))
    off_end_i = off_end.astype(jnp.int32)
    off_end_bc = jnp.broadcast_to(off_end_i, (NTE, N_EXPERTS))
    tile_base = lax.broadcasted_iota(jnp.int32, (NTE, N_EXPERTS), 0) * BT
    te = jnp.sum(jnp.where(off_end_bc <= tile_base, 1, 0), axis=-1, keepdims=True)
    te = jnp.minimum(te, N_EXPERTS - 1)
    n_act = jnp.max(off_end_i, axis=-1, keepdims=True) // BT
    row_i = lax.broadcasted_iota(jnp.int32, (NTE, 1), 0)
    te_ref[...] = jnp.where(row_i < NT, te, jnp.broadcast_to(n_act, (NTE, 1)))


def _router_call(x, router_w):
    return pl.pallas_call(
        _router_body,
        grid=(1,),
        in_specs=[
            pl.BlockSpec((N_TOKENS, D_MODEL), lambda i: (0, 0)),
            pl.BlockSpec((N_EXPERTS, D_MODEL), lambda i: (0, 0)),
        ],
        out_specs=[
            pl.BlockSpec((N_TOKENS, 1), lambda i: (0, 0)),
            pl.BlockSpec((N_TOKENS, 1), lambda i: (0, 0)),
            pl.BlockSpec((N_TOKENS, GW), lambda i: (0, 0)),
            pl.BlockSpec((N_TOKENS, GW), lambda i: (0, 0)),
            pl.BlockSpec((NTE, 1), lambda i: (0, 0)),
        ],
        out_shape=[
            jax.ShapeDtypeStruct((N_TOKENS, 1), jnp.int32),
            jax.ShapeDtypeStruct((N_TOKENS, 1), jnp.int32),
            jax.ShapeDtypeStruct((N_TOKENS, GW), jnp.float32),
            jax.ShapeDtypeStruct((N_TOKENS, GW), jnp.float32),
            jax.ShapeDtypeStruct((NTE, 1), jnp.int32),
        ],
    )(x, router_w)


def _dispatch_body(x_hbm, slot0_hbm, slot1_hbm, g0_hbm, g1_hbm,
                   xd_hbm, gx_hbm,
                   idx0_v, idx1_v, rows_v, ga_v, gb_v, sem, semg):
    wid = lax.axis_index("s") * 2 + lax.axis_index("c")
    base = wid * TPW
    l0 = pltpu.async_copy(slot0_hbm.at[pl.ds(base, TPW)], idx0_v, sem)
    l1 = pltpu.async_copy(slot1_hbm.at[pl.ds(base, TPW)], idx1_v, sem)
    l2 = pltpu.async_copy(x_hbm.at[pl.ds(base, TPW)], rows_v, sem)
    l3 = pltpu.async_copy(g0_hbm.at[pl.ds(base, TPW)], ga_v, sem)
    l4 = pltpu.async_copy(g1_hbm.at[pl.ds(base, TPW)], gb_v, sem)
    l0.wait(); l1.wait(); l2.wait(); l3.wait(); l4.wait()
    cp0 = pltpu.async_copy(rows_v, xd_hbm.at[idx0_v], sem)
    cpg0 = pltpu.async_copy(ga_v, gx_hbm.at[idx0_v], semg)
    cp1 = pltpu.async_copy(rows_v, xd_hbm.at[idx1_v], sem)
    cpg1 = pltpu.async_copy(gb_v, gx_hbm.at[idx1_v], semg)
    cp0.wait(); cpg0.wait(); cp1.wait(); cpg1.wait()


def _dispatch_call(x, slot0, slot1, g0, g1):
    fn = pl.kernel(
        _dispatch_body,
        out_type=[
            jax.ShapeDtypeStruct((S_MAX, D_MODEL), jnp.float32),
            jax.ShapeDtypeStruct((S_MAX, GW), jnp.float32),
        ],
        mesh=plsc.VectorSubcoreMesh(core_axis_name="c", subcore_axis_name="s"),
        scratch_types=[
            pltpu.VMEM((TPW,), jnp.int32),
            pltpu.VMEM((TPW,), jnp.int32),
            pltpu.VMEM((TPW, D_MODEL), jnp.float32),
            pltpu.VMEM((TPW, GW), jnp.float32),
            pltpu.VMEM((TPW, GW), jnp.float32),
            pltpu.SemaphoreType.DMA,
            pltpu.SemaphoreType.DMA,
        ],
    )
    return fn(x, slot0, slot1, g0, g1)


def _ffn_body(te_ref, xd_ref, gx_ref, wg_ref, wu_ref, wd_ref, out_ref):
    i = pl.program_id(0)
    n_act = te_ref[NT]

    @pl.when(i < n_act)
    def _():
        xv = xd_ref[...]
        gate = lax.dot_general(xv, wg_ref[0, 0], (((1,), (1,)), ((), ())))
        up = lax.dot_general(xv, wu_ref[0, 0], (((1,), (1,)), ((), ())))
        h = (up * jax.nn.sigmoid(up)) * gate
        yp = lax.dot_general(h, wd_ref[0], (((1,), (1,)), ((), ())))
        out_ref[...] = yp * gx_ref[:, 0:1]


def _ffn_call(te, xd, gx, wg2, w_down):
    def _ieff(i, te_ref):
        return jnp.minimum(i, jnp.maximum(te_ref[NT] - 1, 0))

    return pl.pallas_call(
        _ffn_body,
        grid_spec=pltpu.PrefetchScalarGridSpec(
            num_scalar_prefetch=1,
            grid=(NT,),
            in_specs=[
                pl.BlockSpec((BT, D_MODEL),
                             lambda i, te_ref: (_ieff(i, te_ref), 0)),
                pl.BlockSpec((BT, GW),
                             lambda i, te_ref: (_ieff(i, te_ref), 0)),
                pl.BlockSpec((1, 1, D_FF, D_MODEL),
                             lambda i, te_ref: (te_ref[_ieff(i, te_ref)], 0, 0, 0)),
                pl.BlockSpec((1, 1, D_FF, D_MODEL),
                             lambda i, te_ref: (te_ref[_ieff(i, te_ref)], 1, 0, 0)),
                pl.BlockSpec((1, D_MODEL, D_FF),
                             lambda i, te_ref: (te_ref[_ieff(i, te_ref)], 0, 0)),
            ],
            out_specs=pl.BlockSpec((BT, D_MODEL), lambda i, te_ref: (i, 0)),
        ),
        out_shape=jax.ShapeDtypeStruct((S_MAX, D_MODEL), jnp.float32),
        compiler_params=pltpu.CompilerParams(
            dimension_semantics=("arbitrary",),
            vmem_limit_bytes=104 * 1024 * 1024,
        ),
    )(te, xd, gx, wg2, wg2, w_down)


CH = 32


def _combine_body(ys_hbm, slot0_hbm, slot1_hbm, y_hbm,
                  idx0_v, idx1_v, rows_a, rows_b, sem_a, sem_b):
    wid = lax.axis_index("s") * 2 + lax.axis_index("c")

    def chunk(c, _):
        base = wid * TPW + c * CH
        pltpu.sync_copy(slot0_hbm.at[pl.ds(base, CH)], idx0_v)
        pltpu.sync_copy(slot1_hbm.at[pl.ds(base, CH)], idx1_v)
        cpa = pltpu.async_copy(ys_hbm.at[idx0_v], rows_a, sem_a)
        cpb = pltpu.async_copy(ys_hbm.at[idx1_v], rows_b, sem_b)
        cpa.wait()
        cpb.wait()

        @plsc.parallel_loop(0, CH, 1, unroll=2)
        def _(r):
            for j in range(D_MODEL // 16):
                rows_a[r, pl.ds(j * 16, 16)] = (
                    rows_a[r, pl.ds(j * 16, 16)] + rows_b[r, pl.ds(j * 16, 16)])

        pltpu.sync_copy(rows_a, y_hbm.at[pl.ds(base, CH)])
        return 0

    lax.fori_loop(0, TPW // CH, chunk, 0)


def _combine_call(ys, slot0, slot1):
    fn = pl.kernel(
        _combine_body,
        out_type=jax.ShapeDtypeStruct((N_TOKENS, D_MODEL), jnp.float32),
        mesh=plsc.VectorSubcoreMesh(core_axis_name="c", subcore_axis_name="s"),
        scratch_types=[
            pltpu.VMEM((CH,), jnp.int32),
            pltpu.VMEM((CH,), jnp.int32),
            pltpu.VMEM((CH, D_MODEL), jnp.float32),
            pltpu.VMEM((CH, D_MODEL), jnp.float32),
            pltpu.SemaphoreType.DMA,
            pltpu.SemaphoreType.DMA,
        ],
    )
    return fn(ys, slot0, slot1)


def kernel(x, router_w, w_gate_up, w_down):
    wg2 = w_gate_up.reshape(N_EXPERTS, 2, D_FF, D_MODEL)
    slot0, slot1, g0, g1, te = _router_call(x, router_w)
    slot0 = slot0.reshape(N_TOKENS)
    slot1 = slot1.reshape(N_TOKENS)
    te = te.reshape(NTE)
    xd, gx = _dispatch_call(x, slot0, slot1, g0, g1)
    ys = _ffn_call(te, xd, gx, wg2, w_down)
    return _combine_call(ys, slot0, slot1)
